# baseline (speedup 1.0000x reference)
import functools

import jax
import jax.numpy as jnp
from jax.experimental import pallas as pl
from jax.experimental.pallas import tpu as pltpu

E = 64
K = 2
C = 160
NC = 1
EPS = 2
EPC = E // (NC * EPS)

_NEG = -3.0e38
_INV_SQRT2 = 0.7071067811865476


def _gelu_exact(v):
    return 0.5 * v * (1.0 + jax.lax.erf(v * _INV_SQRT2))


def _col_cumsum(a):
    n = a.shape[0]
    s = 1
    while s < n:
        shifted = jnp.concatenate(
            [jnp.zeros((s,) + a.shape[1:], a.dtype), a[:-s]], axis=0)
        a = a + shifted
        s *= 2
    return a


def _moe_body(scale_ref, logits_ref, x_ref, w1_ref, w3_ref, w2_ref,
              out_ref, ids_ref, ids_s_ref, wts_s_ref):
    c = pl.program_id(0)
    i = pl.program_id(1)
    T = x_ref.shape[0]

    @pl.when(i == 0)
    def _routing():
        g = logits_ref[...]
        iota_e = jax.lax.broadcasted_iota(jnp.int32, g.shape, 1)
        m1 = jnp.max(g, axis=1, keepdims=True)
        id1 = jnp.min(jnp.where(g == m1, iota_e, E), axis=1)
        g2 = jnp.where(iota_e == id1[:, None], _NEG, g)
        m2 = jnp.max(g2, axis=1, keepdims=True)
        id2 = jnp.min(jnp.where(g2 == m2, iota_e, E), axis=1)
        d = jnp.exp(m2[:, 0] - m1[:, 0])
        w_a = 1.0 / (1.0 + d)
        w_b = d / (1.0 + d)
        scale = scale_ref[0, :]
        sel0 = (iota_e == id1[:, None]).astype(jnp.int32)
        sel1 = (iota_e == id2[:, None]).astype(jnp.int32)
        s1 = jnp.sum(sel0.astype(jnp.float32) * scale[None, :], axis=1)
        s2 = jnp.sum(sel1.astype(jnp.float32) * scale[None, :], axis=1)
        pos0_m = _col_cumsum(sel0) - sel0
        tot0 = jnp.sum(sel0, axis=0, keepdims=True)
        pos1_m = _col_cumsum(sel1) - sel1 + tot0
        pos0 = jnp.sum(pos0_m * sel0, axis=1)
        pos1 = jnp.sum(pos1_m * sel1, axis=1)
        fslot0 = jnp.where(pos0 < C, id1 * C + pos0, -1)
        fslot1 = jnp.where(pos1 < C, id2 * C + pos1, -1)
        ids_ref[0, :] = fslot0
        ids_ref[1, :] = fslot1
        ids_s_ref[:, 0:1] = fslot0[:, None]
        ids_s_ref[:, 1:2] = fslot1[:, None]
        wts_s_ref[:, 0:1] = (w_a * s1)[:, None]
        wts_s_ref[:, 1:2] = (w_b * s2)[:, None]

    fslot0 = ids_ref[0, :]
    fslot1 = ids_ref[1, :]
    fslot0_s = ids_s_ref[:, 0:1]
    fslot1_s = ids_s_ref[:, 1:2]
    wa_s = wts_s_ref[:, 0:1]
    wb_s = wts_s_ref[:, 1:2]

    xb = x_ref[...]
    e0 = (c * EPC + i) * EPS
    CP = EPS * C
    iota_ct = jax.lax.broadcasted_iota(jnp.int32, (CP, T), 0) + e0 * C
    pd_t = ((iota_ct == fslot0[None, :]) |
            (iota_ct == fslot1[None, :])).astype(jnp.bfloat16)
    xep = jax.lax.dot_general(pd_t, xb, (((1,), (0,)), ((), ())),
                              preferred_element_type=jnp.float32)
    yes = []
    for t in range(EPS):
        xe = xep[t * C:(t + 1) * C]
        w1e = w1_ref[t]
        w3e = w3_ref[t]
        w2e = w2_ref[t]
        h1 = jax.lax.dot_general(xe, w1e, (((1,), (1,)), ((), ())),
                                 preferred_element_type=jnp.float32)
        h3 = jax.lax.dot_general(xe, w3e, (((1,), (1,)), ((), ())),
                                 preferred_element_type=jnp.float32)
        h = _gelu_exact(h1) * h3
        ye = jax.lax.dot_general(h, w2e, (((1,), (1,)), ((), ())),
                                 preferred_element_type=jnp.float32)
        yes.append(ye)
    yep = jnp.concatenate(yes, axis=0)

    iota_tc = jax.lax.broadcasted_iota(jnp.int32, (T, CP), 1) + e0 * C
    pw = (jnp.where(iota_tc == fslot0_s, wa_s, 0.0)
          + jnp.where(iota_tc == fslot1_s, wb_s, 0.0)
          ).astype(jnp.bfloat16)
    contrib = jax.lax.dot_general(pw, yep.astype(jnp.bfloat16),
                                  (((1,), (0,)), ((), ())),
                                  preferred_element_type=jnp.float32)

    @pl.when(i == 0)
    def _init():
        out_ref[0] = contrib

    @pl.when(i > 0)
    def _acc():
        out_ref[0] += contrib


def kernel(x, router_logits, per_expert_scale, w1, w3, w2):
    T, D = x.shape
    F = w1.shape[1]
    scale2d = per_expert_scale.reshape(1, E)
    xb16 = x.astype(jnp.bfloat16)

    grid_spec = pltpu.PrefetchScalarGridSpec(
        num_scalar_prefetch=0,
        grid=(NC, EPC),
        in_specs=[
            pl.BlockSpec((1, E), lambda c, i: (0, 0)),
            pl.BlockSpec((T, E), lambda c, i: (0, 0)),
            pl.BlockSpec((T, D), lambda c, i: (0, 0)),
            pl.BlockSpec((EPS, F, D), lambda c, i: (c * EPC + i, 0, 0)),
            pl.BlockSpec((EPS, F, D), lambda c, i: (c * EPC + i, 0, 0)),
            pl.BlockSpec((EPS, D, F), lambda c, i: (c * EPC + i, 0, 0)),
        ],
        out_specs=pl.BlockSpec((1, T, D), lambda c, i: (c, 0, 0)),
        scratch_shapes=[
            pltpu.VMEM((K, T), jnp.int32),
            pltpu.VMEM((T, K), jnp.int32),
            pltpu.VMEM((T, K), jnp.float32),
        ],
    )
    partial_out = pl.pallas_call(
        _moe_body,
        grid_spec=grid_spec,
        out_shape=jax.ShapeDtypeStruct((NC, T, D), jnp.float32),
        compiler_params=pltpu.CompilerParams(
            dimension_semantics=("parallel", "arbitrary"),
        ),
    )(scale2d, router_logits, xb16, w1, w3, w2)
    out = partial_out[0]
    for p in range(1, NC):
        out = out + partial_out[p]
    return out


# speedup vs baseline: 1.0081x; 1.0081x over previous
import functools

import jax
import jax.numpy as jnp
from jax.experimental import pallas as pl
from jax.experimental.pallas import tpu as pltpu

E = 64
K = 2
C = 160
NC = 1
EPS = 2
EPC = E // (NC * EPS)

_NEG = -3.0e38
_INV_SQRT2 = 0.7071067811865476


def _gelu_exact(v):
    return 0.5 * v * (1.0 + jax.lax.erf(v * _INV_SQRT2))


def _col_cumsum(a):
    n = a.shape[0]
    s = 1
    while s < n:
        shifted = jnp.concatenate(
            [jnp.zeros((s,) + a.shape[1:], a.dtype), a[:-s]], axis=0)
        a = a + shifted
        s *= 2
    return a


def _moe_body(scale_ref, logits_ref, x_ref, w1_ref, w3_ref, w2_ref,
              out_ref, ids_ref, ids_s_ref, wts_s_ref):
    c = pl.program_id(0)
    i = pl.program_id(1)
    T = x_ref.shape[0]

    @pl.when(i == 0)
    def _routing():
        g = logits_ref[...]
        iota_e = jax.lax.broadcasted_iota(jnp.int32, g.shape, 1)
        m1 = jnp.max(g, axis=1, keepdims=True)
        id1 = jnp.min(jnp.where(g == m1, iota_e, E), axis=1)
        g2 = jnp.where(iota_e == id1[:, None], _NEG, g)
        m2 = jnp.max(g2, axis=1, keepdims=True)
        id2 = jnp.min(jnp.where(g2 == m2, iota_e, E), axis=1)
        d = jnp.exp(m2[:, 0] - m1[:, 0])
        w_a = 1.0 / (1.0 + d)
        w_b = d / (1.0 + d)
        scale = scale_ref[0, :]
        sel0 = (iota_e == id1[:, None]).astype(jnp.int32)
        sel1 = (iota_e == id2[:, None]).astype(jnp.int32)
        s1 = jnp.sum(sel0.astype(jnp.float32) * scale[None, :], axis=1)
        s2 = jnp.sum(sel1.astype(jnp.float32) * scale[None, :], axis=1)
        pos0_m = _col_cumsum(sel0) - sel0
        tot0 = jnp.sum(sel0, axis=0, keepdims=True)
        pos1_m = _col_cumsum(sel1) - sel1 + tot0
        pos0 = jnp.sum(pos0_m * sel0, axis=1)
        pos1 = jnp.sum(pos1_m * sel1, axis=1)
        fslot0 = jnp.where(pos0 < C, id1 * C + pos0, -1)
        fslot1 = jnp.where(pos1 < C, id2 * C + pos1, -1)
        ids_ref[0, :] = fslot0
        ids_ref[1, :] = fslot1
        ids_s_ref[:, 0:1] = fslot0[:, None]
        ids_s_ref[:, 1:2] = fslot1[:, None]
        wts_s_ref[:, 0:1] = (w_a * s1)[:, None]
        wts_s_ref[:, 1:2] = (w_b * s2)[:, None]

    fslot0 = ids_ref[0, :]
    fslot1 = ids_ref[1, :]
    fslot0_s = ids_s_ref[:, 0:1]
    fslot1_s = ids_s_ref[:, 1:2]
    wa_s = wts_s_ref[:, 0:1]
    wb_s = wts_s_ref[:, 1:2]

    xb = x_ref[...]
    e0 = (c * EPC + i) * EPS
    CP = EPS * C
    iota_ct = jax.lax.broadcasted_iota(jnp.int32, (CP, T), 0) + e0 * C
    pd_t = ((iota_ct == fslot0[None, :]) |
            (iota_ct == fslot1[None, :])).astype(jnp.bfloat16)
    xep = jax.lax.dot_general(pd_t, xb, (((1,), (0,)), ((), ())),
                              preferred_element_type=jnp.float32)
    yes = []
    for t in range(EPS):
        xe = xep[t * C:(t + 1) * C]
        w1e = w1_ref[t]
        w3e = w3_ref[t]
        w2e = w2_ref[t]
        h1 = jax.lax.dot_general(xe, w1e, (((1,), (1,)), ((), ())),
                                 preferred_element_type=jnp.float32)
        h3 = jax.lax.dot_general(xe, w3e, (((1,), (1,)), ((), ())),
                                 preferred_element_type=jnp.float32)
        h = _gelu_exact(h1) * h3
        ye = jax.lax.dot_general(h, w2e, (((1,), (1,)), ((), ())),
                                 preferred_element_type=jnp.float32)
        yes.append(ye)
    yep = jnp.concatenate(yes, axis=0)

    iota_tc = jax.lax.broadcasted_iota(jnp.int32, (T, CP), 1) + e0 * C
    pw = (jnp.where(iota_tc == fslot0_s, wa_s, 0.0)
          + jnp.where(iota_tc == fslot1_s, wb_s, 0.0))
    contrib = jax.lax.dot_general(pw, yep,
                                  (((1,), (0,)), ((), ())),
                                  preferred_element_type=jnp.float32)

    @pl.when(i == 0)
    def _init():
        out_ref[0] = contrib

    @pl.when(i > 0)
    def _acc():
        out_ref[0] += contrib


def kernel(x, router_logits, per_expert_scale, w1, w3, w2):
    T, D = x.shape
    F = w1.shape[1]
    scale2d = per_expert_scale.reshape(1, E)
    xb16 = x.astype(jnp.bfloat16)

    grid_spec = pltpu.PrefetchScalarGridSpec(
        num_scalar_prefetch=0,
        grid=(NC, EPC),
        in_specs=[
            pl.BlockSpec((1, E), lambda c, i: (0, 0)),
            pl.BlockSpec((T, E), lambda c, i: (0, 0)),
            pl.BlockSpec((T, D), lambda c, i: (0, 0)),
            pl.BlockSpec((EPS, F, D), lambda c, i: (c * EPC + i, 0, 0)),
            pl.BlockSpec((EPS, F, D), lambda c, i: (c * EPC + i, 0, 0)),
            pl.BlockSpec((EPS, D, F), lambda c, i: (c * EPC + i, 0, 0)),
        ],
        out_specs=pl.BlockSpec((1, T, D), lambda c, i: (c, 0, 0)),
        scratch_shapes=[
            pltpu.VMEM((K, T), jnp.int32),
            pltpu.VMEM((T, K), jnp.int32),
            pltpu.VMEM((T, K), jnp.float32),
        ],
    )
    partial_out = pl.pallas_call(
        _moe_body,
        grid_spec=grid_spec,
        out_shape=jax.ShapeDtypeStruct((NC, T, D), jnp.float32),
        compiler_params=pltpu.CompilerParams(
            dimension_semantics=("parallel", "arbitrary"),
        ),
    )(scale2d, router_logits, xb16, w1, w3, w2)
    out = partial_out[0]
    for p in range(1, NC):
        out = out + partial_out[p]
    return out
